# Initial kernel scaffold; baseline (speedup 1.0000x reference)
import jax
import jax.numpy as jnp
from jax.experimental import pallas as pl


def kernel(atoms, chirals, bonds, atype_W, chiral_W, apair_W, bond_W):
    raise NotImplementedError("write your pallas kernel here")


# speedup vs baseline: 116.8015x; 116.8015x over previous
import jax
import jax.numpy as jnp
from jax import lax
from jax.experimental import pallas as pl
from jax.experimental.pallas import tpu as pltpu

ATOM_VOC = 128
CHIRAL_PAD = 8
BOND_VOC = 32
D_MODEL = 512
NHEAD = 16
B = 128
L = 128

_NEG_INF = float("-inf")


def _body(atoms_ref, atomsT_ref, chiralsT_ref, bonds_ref, a3p_ref, bwt_ref,
          aW_ref, cW_ref, emb_ref, out_ref):
    f32 = jnp.float32

    arow_l = atomsT_ref[0]
    crow_l = chiralsT_ref[0]
    viota = lax.broadcasted_iota(jnp.int32, (ATOM_VOC, B), 0)
    oha = ((viota == arow_l) & (viota != 0)).astype(f32)
    ciota = lax.broadcasted_iota(jnp.int32, (CHIRAL_PAD, B), 0)
    ohc = ((ciota == crow_l) & (ciota != 0)).astype(f32)
    embA = lax.dot_general(oha, aW_ref[...], (((0,), (0,)), ((), ())),
                           preferred_element_type=f32)
    embC = lax.dot_general(ohc, cW_ref[...], (((0,), (0,)), ((), ())),
                           preferred_element_type=f32)
    emb_ref[0] = embA + embC

    arow = atoms_ref[0]
    viota2 = lax.broadcasted_iota(jnp.int32, (ATOM_VOC, L), 0)
    obt = (viota2 == arow).astype(f32)
    kb = lax.dot_general(a3p_ref[...], obt, (((1,), (0,)), ((), ())),
                         preferred_element_type=f32)
    k3 = kb.reshape(NHEAD, ATOM_VOC, L)

    bonds2 = bonds_ref[0]
    lane = lax.broadcasted_iota(jnp.int32, (NHEAD, ATOM_VOC), 1)
    lut = jnp.where(lane == 0, f32(0.0), bwt_ref[...])
    maskj = arow == 0

    for h in range(NHEAD):
        ap = lax.dot_general(obt, k3[h], (((0,), (0,)), ((), ())),
                             preferred_element_type=f32)
        lut_h = jnp.broadcast_to(lut[h].reshape(1, ATOM_VOC), (L, L))
        bd = jnp.take_along_axis(lut_h, bonds2, axis=1,
                                 mode="promise_in_bounds")
        out_ref[0, h] = jnp.where(maskj, _NEG_INF, ap + bd)


def kernel(atoms, chirals, bonds, atype_W, chiral_W, apair_W, bond_W):
    a3p = jnp.transpose(apair_W.reshape(ATOM_VOC, ATOM_VOC, NHEAD),
                        (2, 1, 0)).reshape(NHEAD * ATOM_VOC, ATOM_VOC)
    bwt = jnp.pad(bond_W.T, ((0, 0), (0, ATOM_VOC - BOND_VOC)))
    cWp = jnp.pad(chiral_W, ((0, CHIRAL_PAD - chiral_W.shape[0]), (0, 0)))
    atoms3 = atoms.reshape(B, 1, L)
    atomsT3 = atoms.T.reshape(L, 1, B)
    chiralsT3 = chirals.T.reshape(L, 1, B)

    grid = (B,)
    emb, apairs = pl.pallas_call(
        _body,
        grid=grid,
        in_specs=[
            pl.BlockSpec((1, 1, L), lambda b: (b, 0, 0)),
            pl.BlockSpec((1, 1, B), lambda b: (b, 0, 0)),
            pl.BlockSpec((1, 1, B), lambda b: (b, 0, 0)),
            pl.BlockSpec((1, L, L), lambda b: (b, 0, 0)),
            pl.BlockSpec((NHEAD * ATOM_VOC, ATOM_VOC), lambda b: (0, 0)),
            pl.BlockSpec((NHEAD, ATOM_VOC), lambda b: (0, 0)),
            pl.BlockSpec((ATOM_VOC, D_MODEL), lambda b: (0, 0)),
            pl.BlockSpec((CHIRAL_PAD, D_MODEL), lambda b: (0, 0)),
        ],
        out_specs=[
            pl.BlockSpec((1, B, D_MODEL), lambda b: (b, 0, 0)),
            pl.BlockSpec((1, NHEAD, L, L), lambda b: (b, 0, 0, 0)),
        ],
        out_shape=[
            jax.ShapeDtypeStruct((L, B, D_MODEL), jnp.float32),
            jax.ShapeDtypeStruct((B, NHEAD, L, L), jnp.float32),
        ],
        compiler_params=pltpu.CompilerParams(
            dimension_semantics=("arbitrary",),
        ),
    )(atoms3, atomsT3, chiralsT3, bonds, a3p, bwt, atype_W, cWp)
    return emb, apairs


# speedup vs baseline: 132.5837x; 1.1351x over previous
import jax
import jax.numpy as jnp
from jax import lax
from jax.experimental import pallas as pl
from jax.experimental.pallas import tpu as pltpu

ATOM_VOC = 128
BOND_VOC = 32
D_MODEL = 512
NHEAD = 16
B = 128
L = 128

_NEG_INF = float("-inf")


def _body(atoms2_ref, atomsF_ref, atomsT_ref, chiralsF_ref, bonds_ref,
          a3p_ref, bwt_ref, wBig_ref, emb_ref, out_ref):
    f32 = jnp.float32
    bf16 = jnp.bfloat16
    pid = pl.program_id(0)
    liota = lax.broadcasted_iota(jnp.int32, (B, ATOM_VOC), 1)

    def colgather(src_ref, idx):
        pidv = jnp.full((B, L), idx, dtype=jnp.int32)
        return jnp.take_along_axis(src_ref[...], pidv, axis=1,
                                   mode="promise_in_bounds")

    ohs = []
    for k in range(2):
        acol_l = colgather(atomsF_ref, 2 * pid + k)
        ccol_l = colgather(chiralsF_ref, 2 * pid + k)
        ohaT = ((liota == acol_l) & (acol_l != 0)).astype(bf16)
        ohcT = ((liota == ccol_l) & (ccol_l != 0)).astype(bf16)
        ohs.append(jnp.concatenate([ohaT, ohcT], axis=1))
    ohBig = jnp.concatenate(ohs, axis=0)
    embres = lax.dot_general(ohBig, wBig_ref[...],
                             (((1,), (0,)), ((), ())),
                             preferred_element_type=f32)
    emb_ref[0] = embres[:B]
    emb_ref[1] = embres[B:]

    arow_cat = atoms2_ref[0]
    viota2 = lax.broadcasted_iota(jnp.int32, (ATOM_VOC, 2 * L), 0)
    obt2 = (viota2 == arow_cat).astype(bf16)

    kb2 = lax.dot_general(a3p_ref[...], obt2, (((1,), (0,)), ((), ())),
                          preferred_element_type=f32)
    k3 = kb2.reshape(NHEAD, ATOM_VOC, 2 * L).astype(bf16)

    lane = lax.broadcasted_iota(jnp.int32, (NHEAD, ATOM_VOC), 1)
    lut = jnp.where(lane == 0, f32(0.0), bwt_ref[...])

    for k in range(2):
        k2 = jnp.concatenate(
            [k3[h][:, k * L:(k + 1) * L] for h in range(NHEAD)], axis=1)
        acol_b = colgather(atomsT_ref, 2 * pid + k)
        obtT = (liota == acol_b).astype(bf16)
        ap_all = lax.dot_general(obtT, k2, (((1,), (0,)), ((), ())),
                                 preferred_element_type=f32)
        bonds2 = bonds_ref[k]
        maskj = arow_cat[:, k * L:(k + 1) * L] == 0
        for h in range(NHEAD):
            lut_h = jnp.broadcast_to(lut[h].reshape(1, ATOM_VOC), (L, L))
            bd = jnp.take_along_axis(lut_h, bonds2, axis=1,
                                     mode="promise_in_bounds")
            ap = ap_all[:, h * L:(h + 1) * L]
            out_ref[k, h] = jnp.where(maskj, _NEG_INF, ap + bd)


def kernel(atoms, chirals, bonds, atype_W, chiral_W, apair_W, bond_W):
    a3p = jnp.transpose(apair_W.reshape(ATOM_VOC, ATOM_VOC, NHEAD),
                        (2, 1, 0)).reshape(NHEAD * ATOM_VOC,
                                           ATOM_VOC).astype(jnp.bfloat16)
    bwt = jnp.pad(bond_W.T, ((0, 0), (0, ATOM_VOC - BOND_VOC)))
    cWp = jnp.pad(chiral_W, ((0, ATOM_VOC - chiral_W.shape[0]), (0, 0)))
    wBig = jnp.concatenate([atype_W, cWp], axis=0).astype(jnp.bfloat16)
    atoms2 = atoms.reshape(B // 2, 1, 2 * L)
    atomsT = atoms.T

    grid = (B // 2,)
    emb, apairs = pl.pallas_call(
        _body,
        grid=grid,
        in_specs=[
            pl.BlockSpec((1, 1, 2 * L), lambda s: (s, 0, 0)),
            pl.BlockSpec((B, L), lambda s: (0, 0)),
            pl.BlockSpec((L, B), lambda s: (0, 0)),
            pl.BlockSpec((B, L), lambda s: (0, 0)),
            pl.BlockSpec((2, L, L), lambda s: (s, 0, 0)),
            pl.BlockSpec((NHEAD * ATOM_VOC, ATOM_VOC), lambda s: (0, 0)),
            pl.BlockSpec((NHEAD, ATOM_VOC), lambda s: (0, 0)),
            pl.BlockSpec((2 * ATOM_VOC, D_MODEL), lambda s: (0, 0)),
        ],
        out_specs=[
            pl.BlockSpec((2, B, D_MODEL), lambda s: (s, 0, 0)),
            pl.BlockSpec((2, NHEAD, L, L), lambda s: (s, 0, 0, 0)),
        ],
        out_shape=[
            jax.ShapeDtypeStruct((L, B, D_MODEL), jnp.float32),
            jax.ShapeDtypeStruct((B, NHEAD, L, L), jnp.float32),
        ],
        compiler_params=pltpu.CompilerParams(
            dimension_semantics=("arbitrary",),
        ),
    )(atoms2, atoms, atomsT, chirals, bonds, a3p, bwt, wBig)
    return emb, apairs
